# baseline (speedup 1.0000x reference)
import functools

import jax
import jax.numpy as jnp
from jax import lax
from jax.experimental import pallas as pl
from jax.experimental.pallas import tpu as pltpu
from jax.experimental.pallas import tpu_sc as plsc

N = 10000
E = 320000
D = 128
T = 4
STEPS = 8
G = 16

NP = 10240
NB = 10
BR = NP // NB

NC = 2
NS = 16
CW = D // NC
CH = 128
K = 8
NGRP = 20
PER_T = NGRP * K * CH
E_PAD = NS * PER_T
ZR = 128
ROWS_PER_TILE = NP // NS
DUMMY_DST = N


def _sc_body(ht_ref, idxp_ref, zrow_ref, out_ref,
             acc_ref, ib_buf, row_buf,
             semi, sem0, sem1, sem2, sem3, sem4, sem5, sem6, sem7):
    c = lax.axis_index("c")
    s = lax.axis_index("s")
    sem = (sem0, sem1, sem2, sem3, sem4, sem5, sem6, sem7)

    for q in range(ROWS_PER_TILE // ZR):
        pltpu.async_copy(
            zrow_ref, acc_ref.at[pl.ds(s * ROWS_PER_TILE + q * ZR, ZR)], sem0)
    for q in range(ROWS_PER_TILE // ZR):
        pltpu.make_async_copy(
            zrow_ref, acc_ref.at[pl.ds(s * ROWS_PER_TILE + q * ZR, ZR)], sem0
        ).wait()
    plsc.subcore_barrier()

    def group(g, carry):
        pltpu.async_copy(idxp_ref.at[c, s * NGRP + g], ib_buf, semi).wait()
        gd = [pltpu.async_copy(ht_ref.at[ib_buf.at[k, 0]], row_buf.at[k],
                               sem[k]) for k in range(K)]
        sd = []
        for k in range(K):
            gd[k].wait()
            sd.append(pltpu.async_copy(row_buf.at[k],
                                       acc_ref.at[ib_buf.at[k, 1]],
                                       sem[k], add=True))
        for k in range(K):
            sd[k].wait()
        return carry

    lax.fori_loop(0, NGRP, group, 0)
    plsc.subcore_barrier()

    pltpu.sync_copy(acc_ref.at[pl.ds(s * ROWS_PER_TILE, ROWS_PER_TILE)],
                    out_ref.at[pl.ds(s * ROWS_PER_TILE, ROWS_PER_TILE), c])


def _make_sc_segsum():
    mesh = plsc.VectorSubcoreMesh(core_axis_name="c", subcore_axis_name="s")
    return pl.kernel(
        _sc_body,
        out_type=jax.ShapeDtypeStruct((NP, NC, CW), jnp.float32),
        mesh=mesh,
        scratch_types=[
            pltpu.VMEM_SHARED((NP, CW), jnp.float32),
            pltpu.VMEM((K, 2, CH), jnp.int32),
            pltpu.VMEM((K, CH, CW), jnp.float32),
        ] + [pltpu.SemaphoreType.DMA] * 9,
        compiler_params=pltpu.CompilerParams(use_tc_tiling_on_sc=False),
    )


def _prep_body(x_ref, wall_ref, ball_ref, ht_ref):
    ht_ref[...] = (jnp.dot(x_ref[...], wall_ref[...],
                           preferred_element_type=jnp.float32) + ball_ref[...])


def _gru(h, a, wih, whh, bih, bhh):
    gi = jnp.dot(a, wih, preferred_element_type=jnp.float32) + bih
    gh = jnp.dot(h, whh, preferred_element_type=jnp.float32) + bhh
    r = jax.nn.sigmoid(gi[:, :D] + gh[:, :D])
    z = jax.nn.sigmoid(gi[:, D:2 * D] + gh[:, D:2 * D])
    n = jnp.tanh(gi[:, 2 * D:] + r * gh[:, 2 * D:])
    return (1.0 - z) * n + z * h


def _step_body(h_ref, a_ref, wih_ref, whh_ref, bih_ref, bhh_ref,
               wall_ref, ball_ref, hnew_ref, htnext_ref):
    hn = _gru(h_ref[...], a_ref[...],
              wih_ref[...], whh_ref[...], bih_ref[...], bhh_ref[...])
    hnew_ref[...] = hn
    htnext_ref[...] = (jnp.dot(hn, wall_ref[...],
                               preferred_element_type=jnp.float32) + ball_ref[...])


def _final_body(h_ref, a_ref, wih_ref, whh_ref, bih_ref, bhh_ref,
                gid_ref, wc_ref, bc_ref, hsum_ref, ggnn_ref):
    i = pl.program_id(0)
    hn = _gru(h_ref[...], a_ref[...],
              wih_ref[...], whh_ref[...], bih_ref[...], bhh_ref[...])
    gid = gid_ref[...].reshape(BR).astype(jnp.int32)
    onehot = (lax.broadcasted_iota(jnp.int32, (G, BR), 0)
              == gid[None, :]).astype(jnp.float32)
    contrib = jnp.dot(onehot, hn, preferred_element_type=jnp.float32)

    @pl.when(i == 0)
    def _():
        hsum_ref[...] = jnp.zeros_like(hsum_ref)

    hsum_ref[...] += contrib

    @pl.when(i == NB - 1)
    def _():
        hs = hsum_ref[...]
        ggnn_ref[...] = (jnp.sum(hs * wc_ref[...], axis=1, keepdims=True)
                         + bc_ref[...])


def _full(i):
    return pl.BlockSpec(None, lambda j: tuple(0 for _ in range(i)))


def kernel(x, edge_index, edge_types, graph_ids, W, b, W_ih, W_hh,
           b_ih, b_hh, W_c, b_c):
    f32 = jnp.float32
    src = edge_index[0].astype(jnp.int32)
    dst = edge_index[1].astype(jnp.int32)
    et = edge_types.astype(jnp.int32)
    gidx = src * T + et
    gidx, dst = lax.sort([gidx, dst], num_keys=1)
    pad = E_PAD - E
    gidx = jnp.concatenate([gidx, jnp.zeros((pad,), jnp.int32)])
    dstp = jnp.concatenate([dst, jnp.full((pad,), DUMMY_DST, jnp.int32)])
    g0 = (2 * gidx).reshape(NS * NGRP, K, CH)
    dd = dstp.reshape(NS * NGRP, K, CH)
    idxp = jnp.stack([jnp.stack([g0, dd], axis=2),
                      jnp.stack([g0 + 1, dd], axis=2)], axis=0)

    x_pad = jnp.concatenate([x, jnp.zeros((NP - N, D), f32)], axis=0)
    gid_pad = jnp.concatenate([graph_ids.astype(f32),
                               jnp.full((NP - N,), 1e6, f32)]).reshape(NP // D, D)

    wall = W.transpose(2, 0, 1).reshape(D, T * D)
    ball = b.reshape(1, T * D)
    wih = W_ih.T
    whh = W_hh.T
    bih = b_ih.reshape(1, 3 * D)
    bhh = b_hh.reshape(1, 3 * D)
    bc = b_c.reshape(1, 1)
    wc = W_c
    zrow = jnp.zeros((ZR, CW), f32)

    sc_segsum = _make_sc_segsum()

    prep = pl.pallas_call(
        _prep_body,
        grid=(NB,),
        in_specs=[pl.BlockSpec((BR, D), lambda i: (i, 0)), _full(2), _full(2)],
        out_specs=pl.BlockSpec((BR, T * D), lambda i: (i, 0)),
        out_shape=jax.ShapeDtypeStruct((NP, T * D), f32),
    )

    step = pl.pallas_call(
        _step_body,
        grid=(NB,),
        in_specs=[
            pl.BlockSpec((BR, D), lambda i: (i, 0)),
            pl.BlockSpec((BR, D), lambda i: (i, 0)),
            _full(2), _full(2), _full(2), _full(2), _full(2), _full(2),
        ],
        out_specs=[
            pl.BlockSpec((BR, D), lambda i: (i, 0)),
            pl.BlockSpec((BR, T * D), lambda i: (i, 0)),
        ],
        out_shape=[
            jax.ShapeDtypeStruct((NP, D), f32),
            jax.ShapeDtypeStruct((NP, T * D), f32),
        ],
    )

    final = pl.pallas_call(
        _final_body,
        grid=(NB,),
        in_specs=[
            pl.BlockSpec((BR, D), lambda i: (i, 0)),
            pl.BlockSpec((BR, D), lambda i: (i, 0)),
            _full(2), _full(2), _full(2), _full(2),
            pl.BlockSpec((BR // D, D), lambda i: (i, 0)),
            _full(2), _full(2),
        ],
        out_specs=[
            pl.BlockSpec((G, D), lambda i: (0, 0)),
            pl.BlockSpec((G, 1), lambda i: (0, 0)),
        ],
        out_shape=[
            jax.ShapeDtypeStruct((G, D), f32),
            jax.ShapeDtypeStruct((G, 1), f32),
        ],
    )

    h = x_pad
    ht = prep(x_pad, wall, ball)
    for s in range(STEPS):
        parts = sc_segsum(ht.reshape(T * NP * NC, CW), idxp, zrow)
        a = parts.reshape(NP, D)
        if s < STEPS - 1:
            h, ht = step(h, a, wih, whh, bih, bhh, wall, ball)
        else:
            h_sum, ggnn = final(h, a, wih, whh, bih, bhh, gid_pad, wc, bc)
    return (ggnn, h_sum)


# speedup vs baseline: 1.7062x; 1.7062x over previous
import functools

import jax
import jax.numpy as jnp
from jax import lax
from jax.experimental import pallas as pl
from jax.experimental.pallas import tpu as pltpu
from jax.experimental.pallas import tpu_sc as plsc

N = 10000
E = 320000
D = 128
T = 4
STEPS = 8
G = 16

NP = 10240
NB = 10
BR = NP // NB

NC = 2
NS = 16
CW = D // NC
CH = 128
K = 8
NGRP = 20
PER_T = NGRP * K * CH
E_PAD = NS * PER_T
ZR = 128
ROWS_PER_TILE = NP // NS
DUMMY_DST = N


def _sc_body(ht_ref, idxp_ref, zrow_ref, out_ref,
             acc_ref, ib_buf, row_buf,
             semi, sem0, sem1, sem2, sem3, sem4, sem5, sem6, sem7):
    c = lax.axis_index("c")
    s = lax.axis_index("s")
    sem = (sem0, sem1, sem2, sem3, sem4, sem5, sem6, sem7)

    for q in range(ROWS_PER_TILE // ZR):
        pltpu.async_copy(
            zrow_ref, acc_ref.at[pl.ds(s * ROWS_PER_TILE + q * ZR, ZR)], sem0)
    for q in range(ROWS_PER_TILE // ZR):
        pltpu.make_async_copy(
            zrow_ref, acc_ref.at[pl.ds(s * ROWS_PER_TILE + q * ZR, ZR)], sem0
        ).wait()
    plsc.subcore_barrier()

    def group(g, carry):
        pltpu.async_copy(idxp_ref.at[c, s * NGRP + g], ib_buf, semi).wait()
        gd = [pltpu.async_copy(ht_ref.at[ib_buf.at[k, 0]], row_buf.at[k],
                               sem[k]) for k in range(K)]
        sd = []
        for k in range(K):
            gd[k].wait()
            sd.append(pltpu.async_copy(row_buf.at[k],
                                       acc_ref.at[ib_buf.at[k, 1]],
                                       sem[k], add=True))
        for k in range(K):
            sd[k].wait()
        return carry

    lax.fori_loop(0, NGRP, group, 0)
    plsc.subcore_barrier()

    pltpu.sync_copy(acc_ref.at[pl.ds(s * ROWS_PER_TILE, ROWS_PER_TILE)],
                    out_ref.at[pl.ds(s * ROWS_PER_TILE, ROWS_PER_TILE), c])


def _make_sc_segsum():
    mesh = plsc.VectorSubcoreMesh(core_axis_name="c", subcore_axis_name="s")
    return pl.kernel(
        _sc_body,
        out_type=jax.ShapeDtypeStruct((NP, NC, CW), jnp.bfloat16),
        mesh=mesh,
        scratch_types=[
            pltpu.VMEM_SHARED((NP, CW), jnp.bfloat16),
            pltpu.VMEM((K, 2, CH), jnp.int32),
            pltpu.VMEM((K, CH, CW), jnp.bfloat16),
        ] + [pltpu.SemaphoreType.DMA] * 9,
        compiler_params=pltpu.CompilerParams(use_tc_tiling_on_sc=False),
    )


def _prep_body(x_ref, wall_ref, ball_ref, ht_ref):
    ht_ref[...] = (jnp.dot(x_ref[...], wall_ref[...],
                           preferred_element_type=jnp.float32)
                   + ball_ref[...]).astype(jnp.bfloat16)


def _gru(h, a, wih, whh, bih, bhh):
    gi = jnp.dot(a, wih, preferred_element_type=jnp.float32) + bih
    gh = jnp.dot(h, whh, preferred_element_type=jnp.float32) + bhh
    r = jax.nn.sigmoid(gi[:, :D] + gh[:, :D])
    z = jax.nn.sigmoid(gi[:, D:2 * D] + gh[:, D:2 * D])
    n = jnp.tanh(gi[:, 2 * D:] + r * gh[:, 2 * D:])
    return (1.0 - z) * n + z * h


def _step_body(h_ref, a_ref, wih_ref, whh_ref, bih_ref, bhh_ref,
               wall_ref, ball_ref, hnew_ref, htnext_ref):
    hn = _gru(h_ref[...], a_ref[...].astype(jnp.float32),
              wih_ref[...], whh_ref[...], bih_ref[...], bhh_ref[...])
    hnew_ref[...] = hn
    htnext_ref[...] = (jnp.dot(hn, wall_ref[...],
                               preferred_element_type=jnp.float32)
                       + ball_ref[...]).astype(jnp.bfloat16)


def _final_body(h_ref, a_ref, wih_ref, whh_ref, bih_ref, bhh_ref,
                gid_ref, wc_ref, bc_ref, hsum_ref, ggnn_ref):
    i = pl.program_id(0)
    hn = _gru(h_ref[...], a_ref[...].astype(jnp.float32),
              wih_ref[...], whh_ref[...], bih_ref[...], bhh_ref[...])
    gid = gid_ref[...].reshape(BR).astype(jnp.int32)
    onehot = (lax.broadcasted_iota(jnp.int32, (G, BR), 0)
              == gid[None, :]).astype(jnp.float32)
    contrib = jnp.dot(onehot, hn, preferred_element_type=jnp.float32)

    @pl.when(i == 0)
    def _():
        hsum_ref[...] = jnp.zeros_like(hsum_ref)

    hsum_ref[...] += contrib

    @pl.when(i == NB - 1)
    def _():
        hs = hsum_ref[...]
        ggnn_ref[...] = (jnp.sum(hs * wc_ref[...], axis=1, keepdims=True)
                         + bc_ref[...])


def _full(i):
    return pl.BlockSpec(None, lambda j: tuple(0 for _ in range(i)))


def kernel(x, edge_index, edge_types, graph_ids, W, b, W_ih, W_hh,
           b_ih, b_hh, W_c, b_c):
    f32 = jnp.float32
    src = edge_index[0].astype(jnp.int32)
    dst = edge_index[1].astype(jnp.int32)
    et = edge_types.astype(jnp.int32)
    gidx = src * T + et
    pad = E_PAD - E
    gidx = jnp.concatenate([gidx, jnp.zeros((pad,), jnp.int32)])
    dstp = jnp.concatenate([dst, jnp.full((pad,), DUMMY_DST, jnp.int32)])
    g0 = (2 * gidx).reshape(NS * NGRP, K, CH)
    dd = dstp.reshape(NS * NGRP, K, CH)
    idxp = jnp.stack([jnp.stack([g0, dd], axis=2),
                      jnp.stack([g0 + 1, dd], axis=2)], axis=0)

    x_pad = jnp.concatenate([x, jnp.zeros((NP - N, D), f32)], axis=0)
    gid_pad = jnp.concatenate([graph_ids.astype(f32),
                               jnp.full((NP - N,), 1e6, f32)]).reshape(NP // D, D)

    wall = W.transpose(2, 0, 1).reshape(D, T * D)
    ball = b.reshape(1, T * D)
    wih = W_ih.T
    whh = W_hh.T
    bih = b_ih.reshape(1, 3 * D)
    bhh = b_hh.reshape(1, 3 * D)
    bc = b_c.reshape(1, 1)
    wc = W_c
    zrow = jnp.zeros((ZR, CW), jnp.bfloat16)

    sc_segsum = _make_sc_segsum()

    prep = pl.pallas_call(
        _prep_body,
        grid=(NB,),
        in_specs=[pl.BlockSpec((BR, D), lambda i: (i, 0)), _full(2), _full(2)],
        out_specs=pl.BlockSpec((BR, T * D), lambda i: (i, 0)),
        out_shape=jax.ShapeDtypeStruct((NP, T * D), jnp.bfloat16),
    )

    step = pl.pallas_call(
        _step_body,
        grid=(NB,),
        in_specs=[
            pl.BlockSpec((BR, D), lambda i: (i, 0)),
            pl.BlockSpec((BR, D), lambda i: (i, 0)),
            _full(2), _full(2), _full(2), _full(2), _full(2), _full(2),
        ],
        out_specs=[
            pl.BlockSpec((BR, D), lambda i: (i, 0)),
            pl.BlockSpec((BR, T * D), lambda i: (i, 0)),
        ],
        out_shape=[
            jax.ShapeDtypeStruct((NP, D), f32),
            jax.ShapeDtypeStruct((NP, T * D), jnp.bfloat16),
        ],
    )

    final = pl.pallas_call(
        _final_body,
        grid=(NB,),
        in_specs=[
            pl.BlockSpec((BR, D), lambda i: (i, 0)),
            pl.BlockSpec((BR, D), lambda i: (i, 0)),
            _full(2), _full(2), _full(2), _full(2),
            pl.BlockSpec((BR // D, D), lambda i: (i, 0)),
            _full(2), _full(2),
        ],
        out_specs=[
            pl.BlockSpec((G, D), lambda i: (0, 0)),
            pl.BlockSpec((G, 1), lambda i: (0, 0)),
        ],
        out_shape=[
            jax.ShapeDtypeStruct((G, D), f32),
            jax.ShapeDtypeStruct((G, 1), f32),
        ],
    )

    h = x_pad
    ht = prep(x_pad, wall, ball)
    for s in range(STEPS):
        parts = sc_segsum(ht.reshape(T * NP * NC, CW), idxp, zrow)
        a = parts.reshape(NP, D)
        if s < STEPS - 1:
            h, ht = step(h, a, wih, whh, bih, bhh, wall, ball)
        else:
            h_sum, ggnn = final(h, a, wih, whh, bih, bhh, gid_pad, wc, bc)
    return (ggnn, h_sum)
